# baseline (device time: 75075 ns/iter reference)
import jax
import jax.numpy as jnp
from jax import lax
from jax.experimental import pallas as pl
from jax.experimental.pallas import tpu as pltpu

T = 2048
D = 4096
V_SHARD = 8192
V_SUB = 2048
VT = 256
NT = V_SUB // VT
TB = 1024
NTB = T // TB

N_STAGES = 3


def kernel(x, W, labels):
    labels_col = labels.reshape(T, 1)
    q = 2 * lax.axis_index("x") + lax.axis_index("z")
    q_arr = jnp.asarray(q, jnp.int32).reshape(1)

    def body(q_ref, x_ref, w_ref, lab_ref, out_ref,
             xbf_ref, s_ref, ll_ref, send_ref, recv_ref,
             send_sem, recv_sems):
        t = pl.program_id(0)
        v = pl.program_id(1)
        my_x = lax.axis_index("x")
        my_y = lax.axis_index("y")
        my_z = lax.axis_index("z")

        @pl.when(v == 0)
        def _cast_x():
            for r in range(0, TB, 256):
                xbf_ref[r:r + 256, :] = (
                    x_ref[r:r + 256, :].astype(jnp.bfloat16))

        @pl.when((t == 0) & (v == 0))
        def _init():
            s_ref[...] = jnp.zeros_like(s_ref)
            ll_ref[...] = jnp.zeros_like(ll_ref)

        wbf = w_ref[...].astype(jnp.bfloat16)
        logits = jnp.dot(xbf_ref[...], wbf,
                         preferred_element_type=jnp.float32)

        rows = pl.ds(t * TB, TB)
        s_ref[rows, :] += jnp.sum(jnp.exp(logits), axis=1, keepdims=True)

        v0 = my_y * V_SHARD + q_ref[0] * V_SUB + v * VT
        vocab_ids = lax.broadcasted_iota(jnp.int32, (TB, VT), 1) + v0
        mask = vocab_ids == lab_ref[...]
        ll_ref[rows, :] += jnp.sum(jnp.where(mask, logits, 0.0),
                                   axis=1, keepdims=True)

        @pl.when((t == NTB - 1) & (v == NT - 1))
        def _allreduce():
            acc = jnp.concatenate(
                [s_ref[...].reshape(1, T), ll_ref[...].reshape(1, T)],
                axis=0)
            partners = [
                (1 - my_x, my_y, my_z),
                (my_x, 1 - my_y, my_z),
                (my_x, my_y, 1 - my_z),
            ]
            for st in range(N_STAGES):
                send_ref[...] = acc
                rdma = pltpu.make_async_remote_copy(
                    src_ref=send_ref,
                    dst_ref=recv_ref.at[st],
                    send_sem=send_sem,
                    recv_sem=recv_sems.at[st],
                    device_id=partners[st],
                    device_id_type=pl.DeviceIdType.MESH,
                )
                rdma.start()
                rdma.wait()
                acc = acc + recv_ref[st]
            out_ref[...] = jnp.log(acc[0:1, :]) - acc[1:2, :]

    grid_spec = pltpu.PrefetchScalarGridSpec(
        num_scalar_prefetch=1,
        grid=(NTB, NT),
        in_specs=[
            pl.BlockSpec((TB, D), lambda t, v, q: (t, 0)),
            pl.BlockSpec((D, VT), lambda t, v, q: (0, q[0] * NT + v)),
            pl.BlockSpec((TB, 1), lambda t, v, q: (t, 0)),
        ],
        out_specs=pl.BlockSpec((1, T), lambda t, v, q: (0, 0)),
        scratch_shapes=[
            pltpu.VMEM((TB, D), jnp.bfloat16),
            pltpu.VMEM((T, 1), jnp.float32),
            pltpu.VMEM((T, 1), jnp.float32),
            pltpu.VMEM((2, T), jnp.float32),
            pltpu.VMEM((N_STAGES, 2, T), jnp.float32),
            pltpu.SemaphoreType.DMA,
            pltpu.SemaphoreType.DMA((N_STAGES,)),
        ],
    )

    out = pl.pallas_call(
        body,
        grid_spec=grid_spec,
        out_shape=jax.ShapeDtypeStruct((1, T), jnp.float32),
        compiler_params=pltpu.CompilerParams(
            vmem_limit_bytes=60 * 1024 * 1024),
    )(q_arr, x, W, labels_col)
    return out.reshape(T)
